# baseline (speedup 1.0000x reference)
import functools

import jax
import jax.numpy as jnp
from jax import lax
from jax.experimental import pallas as pl
from jax.experimental.pallas import tpu as pltpu
from jax.experimental.pallas import tpu_sc as plsc

ROWS = 64
N = 32768
LANES = 16
NV = N // LANES
NB = 256
NW = 32
ROWS_PER_W = ROWS // NW
CH = 4
VPC = NV // CH
SEG = VPC * LANES


def _i32(x):
  return jnp.int32(x)


def _to_key(u):
  m = lax.shift_right_arithmetic(u, 31)
  flip = lax.bitwise_not(lax.bitwise_or(m, _i32(-(2**31))))
  return lax.bitwise_xor(u, flip)


def _sigma(pos):
  j3 = lax.bitwise_and(pos, _i32(-SEG))
  icol = lax.bitwise_and(pos, _i32(VPC - 1))
  l = lax.bitwise_and(lax.shift_right_logical(pos, _i32(9)), _i32(15))
  return lax.bitwise_or(
      lax.bitwise_or(j3, lax.shift_left(icol, _i32(4))), l
  )


@functools.cache
def _build():
  mesh = plsc.VectorSubcoreMesh(core_axis_name="c", subcore_axis_name="s")

  @functools.partial(
      pl.kernel,
      out_type=jax.ShapeDtypeStruct((ROWS, N), jnp.float32),
      mesh=mesh,
      compiler_params=pltpu.CompilerParams(needs_layout_passes=False),
      scratch_types=[
          pltpu.VMEM((N,), jnp.float32),
          pltpu.VMEM((N,), jnp.float32),
          pltpu.VMEM((N,), jnp.float32),
      ] + [pltpu.VMEM((LANES * NB,), jnp.int32) for _ in range(CH)],
  )
  def ranker(seq_hbm, out_hbm, key_ref, bufa, bufb, *hists):
    wid = lax.axis_index("s") * 2 + lax.axis_index("c")
    lanes = lax.iota(jnp.int32, LANES)
    lanebase = lanes * _i32(NB)
    ones = jnp.ones((LANES,), jnp.int32)
    zeros = jnp.zeros((LANES,), jnp.int32)
    rho_base = [lanes * _i32(VPC) + _i32(j * SEG) for j in range(CH)]

    def fetch(p, src, j, i, transform):
      if p == 0:
        rho = rho_base[j] + i
        k = plsc.bitcast(plsc.load_gather(key_ref, [rho]), jnp.int32)
        if transform:
          k = _to_key(k)
          plsc.store_scatter(key_ref, [rho], plsc.bitcast(k, jnp.float32))
        d = lax.bitwise_and(k, _i32(0xFF))
        pk = lax.bitwise_or(
            lax.bitwise_and(lax.shift_right_logical(k, _i32(1)), _i32(-32768)),
            rho,
        )
        return d, pk
      sl = pl.ds((j * VPC + i) * LANES, LANES)
      pk = plsc.bitcast(src[sl], jnp.int32)
      if p == 1:
        idx = lax.bitwise_and(pk, _i32(0x7FFF))
        k = plsc.bitcast(plsc.load_gather(key_ref, [idx]), jnp.int32)
        d = lax.bitwise_and(lax.shift_right_logical(k, _i32(8)), _i32(0xFF))
      else:
        d = lax.bitwise_and(
            lax.shift_right_logical(pk, _i32(15 if p == 2 else 23)),
            _i32(0xFF),
        )
      return d, pk

    def run_pass(p, src, dst):
      @pl.loop(0, LANES * NB // LANES)
      def _clear(i):
        sl = pl.ds(i * LANES, LANES)
        for j in range(CH):
          hists[j][sl] = zeros

      @pl.loop(0, VPC)
      def _histogram(i):
        for j in range(CH):
          d, _ = fetch(p, src, j, i, transform=p == 0)
          plsc.addupdate_scatter(hists[j], [lanebase + d], ones)

      @pl.loop(0, NB // LANES, init_carry=jnp.int32(0))
      def _prefix(i, carry):
        def sl(l):
          return pl.ds(l * NB + i * LANES, LANES)

        total = None
        for j in range(CH):
          for l in range(LANES):
            h = hists[j][sl(l)]
            total = h if total is None else total + h
        c = plsc.cumsum(total)
        run = c - total + carry
        for j in range(CH):
          for l in range(LANES):
            h = hists[j][sl(l)]
            hists[j][sl(l)] = run
            run = run + h
        return carry + jnp.sum(total)

      @pl.loop(0, VPC)
      def _place(i):
        for j in range(CH):
          d, pk = fetch(p, src, j, i, transform=False)
          hidx = lanebase + d
          pos = plsc.load_gather(hists[j], [hidx])
          plsc.addupdate_scatter(hists[j], [hidx], ones)
          if p < 3:
            plsc.store_scatter(dst, [_sigma(pos)], plsc.bitcast(pk, jnp.float32))
          else:
            idx = lax.bitwise_and(pk, _i32(0x7FFF))
            val = (pos + 1).astype(jnp.float32) * jnp.float32(1.0 / N)
            plsc.store_scatter(dst, [idx], val)

    @pl.loop(0, ROWS_PER_W)
    def _row(r):
      row = wid * ROWS_PER_W + r
      pltpu.sync_copy(seq_hbm.at[row], key_ref)
      run_pass(0, None, bufa)
      run_pass(1, bufa, bufb)
      run_pass(2, bufb, bufa)
      run_pass(3, bufa, bufb)
      pltpu.sync_copy(bufb, out_hbm.at[row])

  return ranker


def kernel(sequence):
  return _build()(sequence)


# speedup vs baseline: 2.5448x; 2.5448x over previous
import functools

import jax
import jax.numpy as jnp
from jax import lax
from jax.experimental import pallas as pl
from jax.experimental.pallas import tpu as pltpu
from jax.experimental.pallas import tpu_sc as plsc

ROWS = 64
N = 32768
LANES = 16
NV = N // LANES
NB = 256
NW = 32
ROWS_PER_W = ROWS // NW
CH = 4
VPC = NV // CH
SEG = VPC * LANES


def _i32(x):
  return jnp.int32(x)


def _to_key(u):
  m = lax.shift_right_arithmetic(u, 31)
  flip = lax.bitwise_not(lax.bitwise_or(m, _i32(-(2**31))))
  return lax.bitwise_xor(u, flip)


def _sigma(pos):
  j3 = lax.bitwise_and(pos, _i32(-SEG))
  icol = lax.bitwise_and(pos, _i32(VPC - 1))
  l = lax.bitwise_and(lax.shift_right_logical(pos, _i32(9)), _i32(15))
  return lax.bitwise_or(
      lax.bitwise_or(j3, lax.shift_left(icol, _i32(4))), l
  )


@functools.cache
def _build():
  mesh = plsc.VectorSubcoreMesh(core_axis_name="c", subcore_axis_name="s")

  @functools.partial(
      pl.kernel,
      out_type=jax.ShapeDtypeStruct((ROWS, N), jnp.float32),
      mesh=mesh,
      compiler_params=pltpu.CompilerParams(needs_layout_passes=False),
      scratch_types=[
          pltpu.VMEM((N,), jnp.float32),
          pltpu.VMEM((N,), jnp.float32),
          pltpu.VMEM((N,), jnp.float32),
      ] + [pltpu.VMEM((LANES * NB,), jnp.int32) for _ in range(CH)],
  )
  def ranker(seq_hbm, out_hbm, key_ref, bufa, bufb, *hists):
    wid = lax.axis_index("s") * 2 + lax.axis_index("c")
    lanes = lax.iota(jnp.int32, LANES)
    lanebase = lanes * _i32(NB)
    lanes16 = lanes * _i32(LANES)
    ones = jnp.ones((LANES,), jnp.int32)
    zeros = jnp.zeros((LANES,), jnp.int32)
    rho_base = [lanes * _i32(VPC) + _i32(j * SEG) for j in range(CH)]

    def fetch(p, src, j, i):
      sl = pl.ds((j * VPC + i) * LANES, LANES)
      pk = plsc.bitcast(src[sl], jnp.int32)
      if p == 0:
        d = lax.bitwise_and(pk, _i32(0xFF))
        rho = rho_base[j] + i
        pk = lax.bitwise_or(
            lax.bitwise_and(lax.shift_right_logical(pk, _i32(1)), _i32(-32768)),
            rho,
        )
      elif p == 1:
        idx = lax.bitwise_and(pk, _i32(0x7FFF))
        k = plsc.bitcast(plsc.load_gather(key_ref, [_sigma(idx)]), jnp.int32)
        d = lax.bitwise_and(lax.shift_right_logical(k, _i32(8)), _i32(0xFF))
      else:
        d = lax.bitwise_and(
            lax.shift_right_logical(pk, _i32(15 if p == 2 else 23)),
            _i32(0xFF),
        )
      return d, pk

    def run_pass(p, src, dst):
      @pl.loop(0, LANES * NB // LANES)
      def _clear(i):
        sl = pl.ds(i * LANES, LANES)
        for j in range(CH):
          hists[j][sl] = zeros

      @pl.loop(0, VPC)
      def _histogram(i):
        hidx = []
        for j in range(CH):
          d, _ = fetch(p, src, j, i)
          hidx.append(lanebase + d)
        for j in range(CH):
          plsc.addupdate_scatter(hists[j], [hidx[j]], ones)

      @pl.loop(0, NB // LANES, init_carry=jnp.int32(0))
      def _prefix(i, carry):
        def sl(l):
          return pl.ds(l * NB + i * LANES, LANES)

        total = None
        for j in range(CH):
          for l in range(LANES):
            h = hists[j][sl(l)]
            total = h if total is None else total + h
        c = plsc.cumsum(total)
        run = c - total + carry
        for j in range(CH):
          for l in range(LANES):
            h = hists[j][sl(l)]
            hists[j][sl(l)] = run
            run = run + h
        return carry + jnp.sum(total)

      @pl.loop(0, VPC)
      def _place(i):
        ds, pks, hidx, pos = [], [], [], []
        for j in range(CH):
          d, pk = fetch(p, src, j, i)
          ds.append(d)
          pks.append(pk)
          hidx.append(lanebase + d)
        for j in range(CH):
          pos.append(plsc.load_gather(hists[j], [hidx[j]]))
        for j in range(CH):
          plsc.addupdate_scatter(hists[j], [hidx[j]], ones)
        for j in range(CH):
          if p < 3:
            plsc.store_scatter(
                dst, [_sigma(pos[j])], plsc.bitcast(pks[j], jnp.float32)
            )
          else:
            idx = lax.bitwise_and(pks[j], _i32(0x7FFF))
            val = (pos[j] + 1).astype(jnp.float32) * jnp.float32(1.0 / N)
            plsc.store_scatter(dst, [idx], val)

    @pl.loop(0, ROWS_PER_W)
    def _row(r):
      row = wid * ROWS_PER_W + r
      pltpu.sync_copy(seq_hbm.at[row], bufa)

      @pl.loop(0, NV)
      def _transpose(i):
        base = i * LANES
        k = _to_key(plsc.bitcast(bufa[pl.ds(base, LANES)], jnp.int32))
        s = (
            lax.bitwise_or(
                lax.bitwise_and(base, _i32(-SEG)),
                lax.shift_right_logical(
                    lax.bitwise_and(base, _i32(SEG - 1)), _i32(9)
                ),
            )
            + lax.shift_left(lax.bitwise_and(base, _i32(VPC - LANES)), _i32(4))
        )
        plsc.store_scatter(key_ref, [lanes16 + s], plsc.bitcast(k, jnp.float32))

      run_pass(0, key_ref, bufa)
      run_pass(1, bufa, bufb)
      run_pass(2, bufb, bufa)
      run_pass(3, bufa, bufb)
      pltpu.sync_copy(bufb, out_hbm.at[row])

  return ranker


def kernel(sequence):
  return _build()(sequence)


# speedup vs baseline: 3.1988x; 1.2570x over previous
import functools

import jax
import jax.numpy as jnp
from jax import lax
from jax.experimental import pallas as pl
from jax.experimental.pallas import tpu as pltpu
from jax.experimental.pallas import tpu_sc as plsc

ROWS = 64
N = 32768
LANES = 16
NV = N // LANES
NB = 256
HB = NB + 1
NW = 32
ROWS_PER_W = ROWS // NW
CH = 4
VPC = NV // CH
SEG = VPC * LANES
PLANE = VPC + 1
BUFW = 64 * PLANE


def _i32(x):
  return jnp.int32(x)


def _to_key(u):
  m = lax.shift_right_arithmetic(u, 31)
  flip = lax.bitwise_not(lax.bitwise_or(m, _i32(-(2**31))))
  return lax.bitwise_xor(u, flip)


def _tau(pos):
  return pos + lax.shift_right_logical(pos, _i32(9))


@functools.cache
def _build():
  mesh = plsc.VectorSubcoreMesh(core_axis_name="c", subcore_axis_name="s")

  @functools.partial(
      pl.kernel,
      out_type=jax.ShapeDtypeStruct((ROWS, N), jnp.float32),
      mesh=mesh,
      compiler_params=pltpu.CompilerParams(needs_layout_passes=False),
      scratch_types=[
          pltpu.VMEM((BUFW,), jnp.float32),
          pltpu.VMEM((BUFW,), jnp.float32),
          pltpu.VMEM((BUFW,), jnp.float32),
      ] + [pltpu.VMEM((LANES * HB,), jnp.int32) for _ in range(CH)],
  )
  def ranker(seq_hbm, out_hbm, key_ref, bufa, bufb, *hists):
    wid = lax.axis_index("s") * 2 + lax.axis_index("c")
    lanes = lax.iota(jnp.int32, LANES)
    lanebase = lanes * _i32(HB)
    ones = jnp.ones((LANES,), jnp.int32)
    zeros = jnp.zeros((LANES,), jnp.int32)
    gbase = [lanes * _i32(PLANE) + _i32(j * LANES * PLANE) for j in range(CH)]
    rbase = [lanes * _i32(VPC) + _i32(j * SEG) for j in range(CH)]

    def fetch(p, src, j, i):
      pk = plsc.bitcast(plsc.load_gather(src, [gbase[j] + i]), jnp.int32)
      if p == 0:
        d = lax.bitwise_and(pk, _i32(0xFF))
        pk = lax.bitwise_or(
            lax.bitwise_and(lax.shift_right_logical(pk, _i32(1)), _i32(-32768)),
            rbase[j] + i,
        )
      elif p == 1:
        idx = lax.bitwise_and(pk, _i32(0x7FFF))
        k = plsc.bitcast(plsc.load_gather(key_ref, [_tau(idx)]), jnp.int32)
        d = lax.bitwise_and(lax.shift_right_logical(k, _i32(8)), _i32(0xFF))
      else:
        d = lax.bitwise_and(
            lax.shift_right_logical(pk, _i32(15 if p == 2 else 23)),
            _i32(0xFF),
        )
      return d, pk

    def run_pass(p, src, dst):
      @pl.loop(0, LANES * HB // LANES)
      def _clear(i):
        sl = pl.ds(i * LANES, LANES)
        for j in range(CH):
          hists[j][sl] = zeros

      @pl.loop(0, VPC)
      def _histogram(i):
        hidx = []
        for j in range(CH):
          d, _ = fetch(p, src, j, i)
          hidx.append(lanebase + d)
        for j in range(CH):
          plsc.addupdate_scatter(hists[j], [hidx[j]], ones)

      @pl.loop(0, NB // LANES, init_carry=jnp.int32(0))
      def _prefix(i, carry):
        def sl(l):
          return pl.ds(l * HB + i * LANES, LANES)

        total = None
        for j in range(CH):
          for l in range(LANES):
            h = hists[j][sl(l)]
            total = h if total is None else total + h
        c = plsc.cumsum(total)
        run = c - total + carry
        for j in range(CH):
          for l in range(LANES):
            h = hists[j][sl(l)]
            hists[j][sl(l)] = run
            run = run + h
        return carry + jnp.sum(total)

      @pl.loop(0, VPC)
      def _place(i):
        pks, hidx, pos = [], [], []
        for j in range(CH):
          d, pk = fetch(p, src, j, i)
          pks.append(pk)
          hidx.append(lanebase + d)
        for j in range(CH):
          pos.append(plsc.load_gather(hists[j], [hidx[j]]))
        for j in range(CH):
          plsc.addupdate_scatter(hists[j], [hidx[j]], ones)
        for j in range(CH):
          if p < 3:
            plsc.store_scatter(
                dst, [_tau(pos[j])], plsc.bitcast(pks[j], jnp.float32)
            )
          else:
            idx = lax.bitwise_and(pks[j], _i32(0x7FFF))
            val = (pos[j] + 1).astype(jnp.float32) * jnp.float32(1.0 / N)
            plsc.store_scatter(dst, [idx], val)

    @pl.loop(0, ROWS_PER_W)
    def _row(r):
      row = wid * ROWS_PER_W + r
      pltpu.sync_copy(seq_hbm.at[row], bufa.at[pl.ds(0, N)])

      @pl.loop(0, VPC)
      def _transform(i):
        ks, addr = [], []
        for c in range(CH):
          base = (c * VPC + i) * LANES
          ks.append(_to_key(plsc.bitcast(bufa[pl.ds(base, LANES)], jnp.int32)))
          addr.append(lanes + (base + lax.shift_right_logical(base, _i32(9))))
        for c in range(CH):
          plsc.store_scatter(key_ref, [addr[c]], plsc.bitcast(ks[c], jnp.float32))

      run_pass(0, key_ref, bufa)
      run_pass(1, bufa, bufb)
      run_pass(2, bufb, bufa)
      run_pass(3, bufa, bufb)
      pltpu.sync_copy(bufb.at[pl.ds(0, N)], out_hbm.at[row])

  return ranker


def kernel(sequence):
  return _build()(sequence)
